# baseline (speedup 1.0000x reference)
import dataclasses

import jax
import jax.numpy as jnp
from jax import lax
from jax.experimental import pallas as pl
from jax.experimental.pallas import tpu as pltpu
from jax.experimental.pallas import tpu_sc as plsc

N_NODES = 10000
N_EDGES = 160000
D = 256
NC = 2
NS = 16
NW = NC * NS
L = 16
HN = 5120
AP = 5248
AP2 = 2 * AP
R_PER_SUB2 = AP2 // NS
NP = 10240
K1 = 128
E_SCAN = 5120
NCH1 = E_SCAN // K1
SB = 32
MAXCH = 42
DROP = 1 << 20
PAD_LOCAL = HN + 32
R_TC = 1024

_SC_COMPILER_PARAMS = pltpu.CompilerParams()
if "needs_layout_passes" in pltpu.CompilerParams.__dataclass_fields__:
    _SC_COMPILER_PARAMS = dataclasses.replace(
        _SC_COMPILER_PARAMS, needs_layout_passes=False)

_PERM = []
for _p in range(D):
    _m, _r = divmod(_p, 32)
    _PERM.append(32 * _m + (2 * _r if _r < 16 else 2 * (_r - 16) + 1))


def _make_partition():
    mesh = plsc.VectorSubcoreMesh(core_axis_name="c", subcore_axis_name="s")

    def body(epk, lists, out_w, stage, pb0, pb1, w_part, kbuf, cbuf):
        c = lax.axis_index("c")
        s = lax.axis_index("s")
        w = c * NS + s
        iota = lax.iota(jnp.int32, L)
        planes = (pb0, pb1)

        kbuf[pl.ds(L, L)] = jnp.full((L,), -1, jnp.int32)

        @pl.loop(0, NP // L)
        def _(r):
            w_part[pl.ds(r * L, L)] = jnp.zeros((L,), jnp.float32)

        def scatter_triple(pb, idxv, sv, dv, cv, m):
            ch = lax.shift_right_logical(idxv, 7)
            ln = lax.bitwise_and(idxv, 127)
            plsc.store_scatter(pb, [ch, jnp.zeros((L,), jnp.int32), ln],
                               sv, mask=m)
            plsc.store_scatter(pb, [ch, jnp.ones((L,), jnp.int32), ln],
                               dv, mask=m)
            plsc.store_scatter(pb, [ch, jnp.full((L,), 2, jnp.int32), ln],
                               cv, mask=m)

        def chunk(k, curs):
            pltpu.sync_copy(epk.at[w, k], stage)
            cur0, cur1 = curs
            for g in range(K1 // L):
                sl = pl.ds(g * L, L)
                d = stage[1, sl]
                sv = stage[0, sl]
                cv = stage[2, sl]
                new = []
                for h, cur in ((0, cur0), (1, cur1)):
                    m = (d >= h * HN) & (d < (h + 1) * HN)
                    ones = jnp.where(m, 1, 0)
                    pc = plsc.cumsum(ones)
                    idxv = pc - 1 + cur
                    scatter_triple(planes[h], idxv, sv, d - h * HN, cv, m)
                    new.append(cur + jnp.sum(ones))
                cur0, cur1 = new

                vf = plsc.bitcast(cv, jnp.float32)
                ds_, vs_ = plsc.sort_key_val(d, vf)
                kbuf[pl.ds(0, L)] = ds_
                knext = plsc.load_gather(kbuf, [iota + 1])
                is_last = (ds_ != knext) & (ds_ < N_NODES)
                cum = plsc.cumsum(vs_)
                cbuf[pl.ds(0, L)] = cum
                kprev = plsc.load_gather(kbuf, [jnp.maximum(iota - 1, 0)])
                is_first = (ds_ != kprev) | (iota == 0)
                start = plsc.cummax(jnp.where(is_first, iota, 0))
                pc2 = plsc.load_gather(cbuf, [jnp.maximum(start - 1, 0)])
                prev = jnp.where(start == 0, 0.0, pc2)
                plsc.addupdate_scatter(w_part, [ds_], cum - prev,
                                       mask=is_last)
            return cur0, cur1

        cur0, cur1 = lax.fori_loop(0, NCH1, chunk,
                                   (jnp.int32(0), jnp.int32(0)))

        for h, cur in ((0, cur0), (1, cur1)):
            for t in range(K1 // L):
                idxv = cur + t * L + iota
                scatter_triple(planes[h], idxv,
                               jnp.zeros((L,), jnp.int32),
                               jnp.full((L,), PAD_LOCAL, jnp.int32),
                               jnp.zeros((L,), jnp.int32), None)
            nch = (cur + K1 - 1) // K1
            plsc.store_scatter(planes[h],
                               [jnp.zeros((L,), jnp.int32),
                                jnp.full((L,), 3, jnp.int32), iota],
                               jnp.where(iota == 0, nch, 0))
            pltpu.sync_copy(planes[h], lists.at[w, h])

        pltpu.sync_copy(w_part, out_w.at[w])

    return pl.kernel(
        body,
        out_type=[
            jax.ShapeDtypeStruct((NW, NC, MAXCH, 4, K1), jnp.int32),
            jax.ShapeDtypeStruct((NW, NP), jnp.float32),
        ],
        mesh=mesh,
        scratch_types=[
            pltpu.VMEM((3, K1), jnp.int32),
            pltpu.VMEM((MAXCH, 4, K1), jnp.int32),
            pltpu.VMEM((MAXCH, 4, K1), jnp.int32),
            pltpu.VMEM((NP,), jnp.float32),
            pltpu.VMEM((2 * L,), jnp.int32),
            pltpu.VMEM((L,), jnp.float32),
        ],
        compiler_params=_SC_COMPILER_PARAMS,
    )


def _make_sc_aggregate():
    mesh = plsc.VectorSubcoreMesh(core_axis_name="c", subcore_axis_name="s")

    def body(tab, lists, out, acc, sts0, sts1, dstc_a, dstc_b,
             rows_i0, rows_i1, rows_fa, rows_fb,
             isem0, isem1, gsem0, gsem1, ssem_a, ssem_b):
        sts = (sts0, sts1)
        rows_i = (rows_i0, rows_i1)
        isem = (isem0, isem1)
        gsem = (gsem0, gsem1)
        c = lax.axis_index("c")
        s = lax.axis_index("s")
        iota = lax.iota(jnp.int32, L)

        @pl.loop(0, K1)
        def _(r):
            for j in range(K1 // L):
                rows_fa[r, pl.ds(j * L, L)] = jnp.zeros((L,), jnp.float32)

        base = s * R_PER_SUB2
        for i in range(5):
            pltpu.sync_copy(rows_fa, acc.at[pl.ds(base + i * K1, K1)])
        pltpu.sync_copy(rows_fa.at[pl.ds(0, 16)],
                        acc.at[pl.ds(base + 5 * K1, 16)])
        plsc.subcore_barrier()

        def stage_idx(wsel, j, b):
            pltpu.async_copy(lists.at[wsel, c, j], sts[b], isem[b])

        def wait_idx(b):
            pltpu.make_async_copy(lists.at[0, 0, 0], sts[b], isem[b]).wait()

        def fire_gather(b, q, p):
            pltpu.async_copy(tab.at[sts[b].at[0, pl.ds(q * SB, SB)]],
                             rows_i[p], gsem[p])

        def wait_gather(b, q, p):
            pltpu.make_async_copy(tab.at[sts[b].at[0, pl.ds(q * SB, SB)]],
                                  rows_i[p], gsem[p]).wait()

        def fire_scatter():
            pltpu.async_copy(rows_fa, acc.at[dstc_a.at[0]], ssem_a,
                             add=True)
            pltpu.async_copy(rows_fb, acc.at[dstc_b.at[0]], ssem_b,
                             add=True)

        def wait_scatter():
            pltpu.make_async_copy(rows_fa, acc.at[dstc_a.at[0]],
                                  ssem_a).wait()
            pltpu.make_async_copy(rows_fb, acc.at[dstc_b.at[0]],
                                  ssem_b).wait()

        def scale_sub(b, q, p):
            @plsc.parallel_loop(0, SB, step=8, unroll=1)
            def _(e):
                for u in range(8):
                    ee = e + u
                    ce = plsc.bitcast(
                        plsc.load_gather(
                            sts[b],
                            [jnp.full((L,), 2, jnp.int32),
                             jnp.full((L,), q * SB + ee, jnp.int32)]),
                        jnp.float32)
                    for m in range(D // 32):
                        iv = rows_i[p][ee, pl.ds(m * L, L)]
                        bv = plsc.bitcast(iv, jnp.bfloat16)
                        av, bv2 = plsc.unpack(
                            bv, format=plsc.PackFormat.INTERLEAVED)
                        tgt = rows_fa if m < 4 else rows_fb
                        mm = m % 4
                        tgt[q * SB + ee, pl.ds(mm * 32, L)] = av * ce
                        tgt[q * SB + ee, pl.ds(mm * 32 + L, L)] = bv2 * ce

        def consume(j, b):
            fire_gather(b, 0, 0)
            fire_gather(b, 1, 1)

            @pl.when(j >= 1)
            def _():
                wait_scatter()

            wait_gather(b, 0, 0)
            scale_sub(b, 0, 0)
            fire_gather(b, 2, 0)
            wait_gather(b, 1, 1)
            scale_sub(b, 1, 1)
            fire_gather(b, 3, 1)
            wait_gather(b, 2, 0)
            scale_sub(b, 2, 0)
            wait_gather(b, 3, 1)
            scale_sub(b, 3, 1)

            for g in range(K1 // L):
                sl = pl.ds(g * L, L)
                dloc2 = sts[b][1, sl] * 2
                dstc_a[0, sl] = dloc2
                dstc_b[0, sl] = dloc2 + 1
            fire_scatter()

        def run_list(wsel):
            stage_idx(wsel, jnp.int32(0), 0)
            wait_idx(0)
            cv = sts0[3, pl.ds(0, L)]
            n = jnp.sum(jnp.where(iota == 0, cv, 0))
            stage_idx(wsel, jnp.int32(1), 1)

            @pl.loop(0, n)
            def _(j):
                b = lax.rem(j, 2)

                @pl.when(b == 0)
                def _():
                    @pl.when(j >= 1)
                    def _():
                        wait_idx(0)
                    consume(j, 0)

                    @pl.when(j + 2 < n)
                    def _():
                        stage_idx(wsel, j + 2, 0)

                @pl.when(b == 1)
                def _():
                    wait_idx(1)
                    consume(j, 1)

                    @pl.when(j + 2 < n)
                    def _():
                        stage_idx(wsel, j + 2, 1)

            @pl.when(n >= 1)
            def _():
                wait_scatter()

            @pl.when(n <= 1)
            def _():
                wait_idx(1)

        run_list(2 * s)
        run_list(2 * s + 1)

        plsc.subcore_barrier()
        for i in range(5):
            pltpu.sync_copy(acc.at[pl.ds(base + i * K1, K1)], rows_fa)
            pltpu.sync_copy(rows_fa, out.at[c, pl.ds(base + i * K1, K1)])
        pltpu.sync_copy(acc.at[pl.ds(base + 5 * K1, 16)],
                        rows_fa.at[pl.ds(0, 16)])
        pltpu.sync_copy(rows_fa.at[pl.ds(0, 16)],
                        out.at[c, pl.ds(base + 5 * K1, 16)])

    return pl.kernel(
        body,
        out_type=jax.ShapeDtypeStruct((NC, AP2, K1), jnp.float32),
        mesh=mesh,
        scratch_types=[
            pltpu.VMEM_SHARED((AP2, K1), jnp.float32),
            pltpu.VMEM((4, K1), jnp.int32),
            pltpu.VMEM((4, K1), jnp.int32),
            pltpu.VMEM((1, K1), jnp.int32),
            pltpu.VMEM((1, K1), jnp.int32),
            pltpu.VMEM((SB, D // 2), jnp.int32),
            pltpu.VMEM((SB, D // 2), jnp.int32),
            pltpu.VMEM((K1, K1), jnp.float32),
            pltpu.VMEM((K1, K1), jnp.float32),
            pltpu.SemaphoreType.DMA,
            pltpu.SemaphoreType.DMA,
            pltpu.SemaphoreType.DMA,
            pltpu.SemaphoreType.DMA,
            pltpu.SemaphoreType.DMA,
            pltpu.SemaphoreType.DMA,
        ],
        compiler_params=_SC_COMPILER_PARAMS,
    )


def _dense(g, w, h, wm, b):
    def body(g_ref, w_ref, h_ref, wm_ref, b_ref, o_ref):
        wsum = jnp.sum(w_ref[...], axis=0)
        inv = 1.0 / jnp.maximum(wsum, 1.0)
        n = jnp.reshape(g_ref[0], (R_TC, D)) * inv[:, None]
        wmat = wm_ref[...]
        z = (jnp.dot(n, wmat[:D], preferred_element_type=jnp.float32)
             + jnp.dot(h_ref[...], wmat[D:],
                       preferred_element_type=jnp.float32))
        z = jnp.maximum(z + b_ref[0], 0.0)
        nrm = jnp.sqrt(jnp.sum(z * z, axis=1, keepdims=True))
        nrm = jnp.where(nrm == 0.0, 1.0, nrm)
        o_ref[...] = z / nrm

    nhb = HN // R_TC
    f = pl.pallas_call(
        body,
        grid=(NC * nhb,),
        in_specs=[
            pl.BlockSpec((1, 2 * R_TC, K1), lambda i: (i // 5, i % 5, 0)),
            pl.BlockSpec((NW, R_TC), lambda i: (0, i)),
            pl.BlockSpec((R_TC, D), lambda i: (i, 0)),
            pl.BlockSpec((2 * D, D), lambda i: (0, 0)),
            pl.BlockSpec((1, D), lambda i: (0, 0)),
        ],
        out_specs=pl.BlockSpec((R_TC, D), lambda i: (i, 0)),
        out_shape=jax.ShapeDtypeStruct((N_NODES, D), jnp.float32),
    )
    return f(g, w, h, wm, b)


_partition = _make_partition()
_sc_agg = _make_sc_aggregate()


def _pack_table(h):
    hb = h.astype(jnp.bfloat16)
    return lax.bitcast_convert_type(hb.reshape(N_NODES, D // 2, 2),
                                    jnp.int32)


def _permute_w(wmat):
    perm = jnp.array(_PERM, dtype=jnp.int32)
    return jnp.concatenate([wmat[:D][perm], wmat[D:]], axis=0)


def kernel(x, edge_index, edge_count, W1, b1, W2, b2):
    epr = N_EDGES // NW
    src = edge_index[0].astype(jnp.int32).reshape(NW, epr)
    dst = edge_index[1].astype(jnp.int32).reshape(NW, epr)
    cnt = edge_count.astype(jnp.float32).reshape(NW, epr)
    pad = E_SCAN - epr
    src = jnp.pad(src, ((0, 0), (0, pad)))
    dst = jnp.pad(dst, ((0, 0), (0, pad)), constant_values=DROP)
    cnt_bits = lax.bitcast_convert_type(
        jnp.pad(cnt, ((0, 0), (0, pad))), jnp.int32)
    epk = jnp.stack([src.reshape(NW, NCH1, K1),
                     dst.reshape(NW, NCH1, K1),
                     cnt_bits.reshape(NW, NCH1, K1)], axis=2)

    lists, w = _partition(epk)

    agg1 = _sc_agg(_pack_table(x), lists)
    h1 = _dense(agg1, w, x, _permute_w(W1), b1.reshape(1, D))

    agg2 = _sc_agg(_pack_table(h1), lists)
    h2 = _dense(agg2, w, h1, _permute_w(W2), b2.reshape(1, D))
    return h2


# speedup vs baseline: 1.1892x; 1.1892x over previous
import dataclasses

import jax
import jax.numpy as jnp
from jax import lax
from jax.experimental import pallas as pl
from jax.experimental.pallas import tpu as pltpu
from jax.experimental.pallas import tpu_sc as plsc

N_NODES = 10000
N_EDGES = 160000
D = 256
NC = 2
NS = 16
NW = NC * NS
L = 16
HN = 5120
AP = 5248
AP2 = 2 * AP
R_PER_SUB2 = AP2 // NS
NP = 10240
K1 = 128
E_SCAN = 5120
NCH1 = E_SCAN // K1
SB = 32
MAXCH = 42
DROP = 1 << 20
PAD_LOCAL = HN + 32
R_TC = 1024

_SC_COMPILER_PARAMS = pltpu.CompilerParams()
if "needs_layout_passes" in pltpu.CompilerParams.__dataclass_fields__:
    _SC_COMPILER_PARAMS = dataclasses.replace(
        _SC_COMPILER_PARAMS, needs_layout_passes=False)

_PERM = []
for _p in range(D):
    _m, _r = divmod(_p, 32)
    _PERM.append(32 * _m + (2 * _r if _r < 16 else 2 * (_r - 16) + 1))


def _make_partition():
    mesh = plsc.VectorSubcoreMesh(core_axis_name="c", subcore_axis_name="s")

    def body(epk, lists, out_w, stage, pb0, pb1, w_part, kbuf, cbuf):
        c = lax.axis_index("c")
        s = lax.axis_index("s")
        w = c * NS + s
        iota = lax.iota(jnp.int32, L)
        planes = (pb0, pb1)

        kbuf[pl.ds(L, L)] = jnp.full((L,), -1, jnp.int32)

        @pl.loop(0, NP // L)
        def _(r):
            w_part[pl.ds(r * L, L)] = jnp.zeros((L,), jnp.float32)

        def scatter_triple(pb, idxv, sv, dv, cv, m):
            ch = lax.shift_right_logical(idxv, 7)
            ln = lax.bitwise_and(idxv, 127)
            plsc.store_scatter(pb, [ch, jnp.zeros((L,), jnp.int32), ln],
                               sv, mask=m)
            plsc.store_scatter(pb, [ch, jnp.ones((L,), jnp.int32), ln],
                               dv, mask=m)
            plsc.store_scatter(pb, [ch, jnp.full((L,), 2, jnp.int32), ln],
                               cv, mask=m)

        def chunk(k, curs):
            pltpu.sync_copy(epk.at[w, k], stage)
            cur0, cur1 = curs
            for g in range(K1 // L):
                sl = pl.ds(g * L, L)
                d = stage[1, sl]
                sv = stage[0, sl]
                cv = stage[2, sl]
                new = []
                for h, cur in ((0, cur0), (1, cur1)):
                    m = (d >= h * HN) & (d < (h + 1) * HN)
                    ones = jnp.where(m, 1, 0)
                    pc = plsc.cumsum(ones)
                    idxv = pc - 1 + cur
                    scatter_triple(planes[h], idxv, sv, d - h * HN, cv, m)
                    new.append(cur + jnp.sum(ones))
                cur0, cur1 = new

                vf = plsc.bitcast(cv, jnp.float32)
                ds_, vs_ = plsc.sort_key_val(d, vf)
                kbuf[pl.ds(0, L)] = ds_
                knext = plsc.load_gather(kbuf, [iota + 1])
                is_last = (ds_ != knext) & (ds_ < N_NODES)
                cum = plsc.cumsum(vs_)
                cbuf[pl.ds(0, L)] = cum
                kprev = plsc.load_gather(kbuf, [jnp.maximum(iota - 1, 0)])
                is_first = (ds_ != kprev) | (iota == 0)
                start = plsc.cummax(jnp.where(is_first, iota, 0))
                pc2 = plsc.load_gather(cbuf, [jnp.maximum(start - 1, 0)])
                prev = jnp.where(start == 0, 0.0, pc2)
                plsc.addupdate_scatter(w_part, [ds_], cum - prev,
                                       mask=is_last)
            return cur0, cur1

        cur0, cur1 = lax.fori_loop(0, NCH1, chunk,
                                   (jnp.int32(0), jnp.int32(0)))

        for h, cur in ((0, cur0), (1, cur1)):
            for t in range(K1 // L):
                idxv = cur + t * L + iota
                scatter_triple(planes[h], idxv,
                               jnp.zeros((L,), jnp.int32),
                               jnp.full((L,), PAD_LOCAL, jnp.int32),
                               jnp.zeros((L,), jnp.int32), None)
            nch = (cur + K1 - 1) // K1
            plsc.store_scatter(planes[h],
                               [jnp.zeros((L,), jnp.int32),
                                jnp.full((L,), 3, jnp.int32), iota],
                               jnp.where(iota == 0, nch, 0))
            pltpu.sync_copy(planes[h], lists.at[w, h])

        pltpu.sync_copy(w_part, out_w.at[w])

    return pl.kernel(
        body,
        out_type=[
            jax.ShapeDtypeStruct((NW, NC, MAXCH, 4, K1), jnp.int32),
            jax.ShapeDtypeStruct((NW, NP), jnp.float32),
        ],
        mesh=mesh,
        scratch_types=[
            pltpu.VMEM((3, K1), jnp.int32),
            pltpu.VMEM((MAXCH, 4, K1), jnp.int32),
            pltpu.VMEM((MAXCH, 4, K1), jnp.int32),
            pltpu.VMEM((NP,), jnp.float32),
            pltpu.VMEM((2 * L,), jnp.int32),
            pltpu.VMEM((L,), jnp.float32),
        ],
        compiler_params=_SC_COMPILER_PARAMS,
    )


def _make_sc_aggregate():
    mesh = plsc.VectorSubcoreMesh(core_axis_name="c", subcore_axis_name="s")

    def body(tab, lists, out, acc, sts0, sts1, dstc_a, dstc_b,
             rows_i0, rows_i1, rows_fa, rows_fb,
             isem0, isem1, gsem0, gsem1, ssem_a, ssem_b):
        sts = (sts0, sts1)
        rows_i = (rows_i0, rows_i1)
        isem = (isem0, isem1)
        gsem = (gsem0, gsem1)
        c = lax.axis_index("c")
        s = lax.axis_index("s")
        iota = lax.iota(jnp.int32, L)

        @pl.loop(0, K1)
        def _(r):
            for j in range(K1 // L):
                rows_fa[r, pl.ds(j * L, L)] = jnp.zeros((L,), jnp.float32)

        base = s * R_PER_SUB2
        for i in range(5):
            pltpu.sync_copy(rows_fa, acc.at[pl.ds(base + i * K1, K1)])
        pltpu.sync_copy(rows_fa.at[pl.ds(0, 16)],
                        acc.at[pl.ds(base + 5 * K1, 16)])
        plsc.subcore_barrier()

        def stage_idx(wsel, j, b):
            pltpu.async_copy(lists.at[wsel, c, j], sts[b], isem[b])

        def wait_idx(b):
            pltpu.make_async_copy(lists.at[0, 0, 0], sts[b], isem[b]).wait()

        def fire_gather(b, q, p):
            pltpu.async_copy(tab.at[sts[b].at[0, pl.ds(q * SB, SB)]],
                             rows_i[p], gsem[p])

        def wait_gather(b, q, p):
            pltpu.make_async_copy(tab.at[sts[b].at[0, pl.ds(q * SB, SB)]],
                                  rows_i[p], gsem[p]).wait()

        def fire_scatter():
            pltpu.async_copy(rows_fa, acc.at[dstc_a.at[0]], ssem_a,
                             add=True)
            pltpu.async_copy(rows_fb, acc.at[dstc_b.at[0]], ssem_b,
                             add=True)

        def wait_scatter():
            pltpu.make_async_copy(rows_fa, acc.at[dstc_a.at[0]],
                                  ssem_a).wait()
            pltpu.make_async_copy(rows_fb, acc.at[dstc_b.at[0]],
                                  ssem_b).wait()

        def scale_sub(b, q, p):
            @plsc.parallel_loop(0, SB, step=4, unroll=2)
            def _(e):
                for u in range(4):
                    ee = e + u
                    ce = plsc.bitcast(
                        plsc.load_gather(
                            sts[b],
                            [jnp.full((L,), 2, jnp.int32),
                             jnp.full((L,), q * SB + ee, jnp.int32)]),
                        jnp.float32)
                    for m in range(D // 32):
                        iv = rows_i[p][ee, pl.ds(m * L, L)]
                        bv = plsc.bitcast(iv, jnp.bfloat16)
                        av, bv2 = plsc.unpack(
                            bv, format=plsc.PackFormat.INTERLEAVED)
                        tgt = rows_fa if m < 4 else rows_fb
                        mm = m % 4
                        tgt[q * SB + ee, pl.ds(mm * 32, L)] = av * ce
                        tgt[q * SB + ee, pl.ds(mm * 32 + L, L)] = bv2 * ce

        def consume(j, b):
            fire_gather(b, 0, 0)
            fire_gather(b, 1, 1)

            @pl.when(j >= 1)
            def _():
                wait_scatter()

            wait_gather(b, 0, 0)
            scale_sub(b, 0, 0)
            fire_gather(b, 2, 0)
            wait_gather(b, 1, 1)
            scale_sub(b, 1, 1)
            fire_gather(b, 3, 1)
            wait_gather(b, 2, 0)
            scale_sub(b, 2, 0)
            wait_gather(b, 3, 1)
            scale_sub(b, 3, 1)

            for g in range(K1 // L):
                sl = pl.ds(g * L, L)
                dloc2 = sts[b][1, sl] * 2
                dstc_a[0, sl] = dloc2
                dstc_b[0, sl] = dloc2 + 1
            fire_scatter()

        def run_list(wsel):
            stage_idx(wsel, jnp.int32(0), 0)
            wait_idx(0)
            cv = sts0[3, pl.ds(0, L)]
            n = jnp.sum(jnp.where(iota == 0, cv, 0))
            stage_idx(wsel, jnp.int32(1), 1)

            @pl.loop(0, n)
            def _(j):
                b = lax.rem(j, 2)

                @pl.when(b == 0)
                def _():
                    @pl.when(j >= 1)
                    def _():
                        wait_idx(0)
                    consume(j, 0)

                    @pl.when(j + 2 < n)
                    def _():
                        stage_idx(wsel, j + 2, 0)

                @pl.when(b == 1)
                def _():
                    wait_idx(1)
                    consume(j, 1)

                    @pl.when(j + 2 < n)
                    def _():
                        stage_idx(wsel, j + 2, 1)

            @pl.when(n >= 1)
            def _():
                wait_scatter()

            @pl.when(n <= 1)
            def _():
                wait_idx(1)

        run_list(2 * s)
        run_list(2 * s + 1)

        plsc.subcore_barrier()
        for i in range(5):
            pltpu.sync_copy(acc.at[pl.ds(base + i * K1, K1)], rows_fa)
            pltpu.sync_copy(rows_fa, out.at[c, pl.ds(base + i * K1, K1)])
        pltpu.sync_copy(acc.at[pl.ds(base + 5 * K1, 16)],
                        rows_fa.at[pl.ds(0, 16)])
        pltpu.sync_copy(rows_fa.at[pl.ds(0, 16)],
                        out.at[c, pl.ds(base + 5 * K1, 16)])

    return pl.kernel(
        body,
        out_type=jax.ShapeDtypeStruct((NC, AP2, K1), jnp.float32),
        mesh=mesh,
        scratch_types=[
            pltpu.VMEM_SHARED((AP2, K1), jnp.float32),
            pltpu.VMEM((4, K1), jnp.int32),
            pltpu.VMEM((4, K1), jnp.int32),
            pltpu.VMEM((1, K1), jnp.int32),
            pltpu.VMEM((1, K1), jnp.int32),
            pltpu.VMEM((SB, D // 2), jnp.int32),
            pltpu.VMEM((SB, D // 2), jnp.int32),
            pltpu.VMEM((K1, K1), jnp.float32),
            pltpu.VMEM((K1, K1), jnp.float32),
            pltpu.SemaphoreType.DMA,
            pltpu.SemaphoreType.DMA,
            pltpu.SemaphoreType.DMA,
            pltpu.SemaphoreType.DMA,
            pltpu.SemaphoreType.DMA,
            pltpu.SemaphoreType.DMA,
        ],
        compiler_params=_SC_COMPILER_PARAMS,
    )


def _dense(g, w, h, wm, b):
    def body(g_ref, w_ref, h_ref, wm_ref, b_ref, o_ref):
        wsum = jnp.sum(w_ref[...], axis=0)
        inv = 1.0 / jnp.maximum(wsum, 1.0)
        n = jnp.reshape(g_ref[0], (R_TC, D)) * inv[:, None]
        wmat = wm_ref[...]
        z = (jnp.dot(n, wmat[:D], preferred_element_type=jnp.float32)
             + jnp.dot(h_ref[...], wmat[D:],
                       preferred_element_type=jnp.float32))
        z = jnp.maximum(z + b_ref[0], 0.0)
        nrm = jnp.sqrt(jnp.sum(z * z, axis=1, keepdims=True))
        nrm = jnp.where(nrm == 0.0, 1.0, nrm)
        o_ref[...] = z / nrm

    nhb = HN // R_TC
    f = pl.pallas_call(
        body,
        grid=(NC * nhb,),
        in_specs=[
            pl.BlockSpec((1, 2 * R_TC, K1), lambda i: (i // 5, i % 5, 0)),
            pl.BlockSpec((NW, R_TC), lambda i: (0, i)),
            pl.BlockSpec((R_TC, D), lambda i: (i, 0)),
            pl.BlockSpec((2 * D, D), lambda i: (0, 0)),
            pl.BlockSpec((1, D), lambda i: (0, 0)),
        ],
        out_specs=pl.BlockSpec((R_TC, D), lambda i: (i, 0)),
        out_shape=jax.ShapeDtypeStruct((N_NODES, D), jnp.float32),
    )
    return f(g, w, h, wm, b)


_partition = _make_partition()
_sc_agg = _make_sc_aggregate()


def _pack_table(h):
    hb = h.astype(jnp.bfloat16)
    return lax.bitcast_convert_type(hb.reshape(N_NODES, D // 2, 2),
                                    jnp.int32)


def _permute_w(wmat):
    perm = jnp.array(_PERM, dtype=jnp.int32)
    return jnp.concatenate([wmat[:D][perm], wmat[D:]], axis=0)


def kernel(x, edge_index, edge_count, W1, b1, W2, b2):
    epr = N_EDGES // NW
    src = edge_index[0].astype(jnp.int32).reshape(NW, epr)
    dst = edge_index[1].astype(jnp.int32).reshape(NW, epr)
    cnt = edge_count.astype(jnp.float32).reshape(NW, epr)
    pad = E_SCAN - epr
    src = jnp.pad(src, ((0, 0), (0, pad)))
    dst = jnp.pad(dst, ((0, 0), (0, pad)), constant_values=DROP)
    cnt_bits = lax.bitcast_convert_type(
        jnp.pad(cnt, ((0, 0), (0, pad))), jnp.int32)
    epk = jnp.stack([src.reshape(NW, NCH1, K1),
                     dst.reshape(NW, NCH1, K1),
                     cnt_bits.reshape(NW, NCH1, K1)], axis=2)

    lists, w = _partition(epk)

    agg1 = _sc_agg(_pack_table(x), lists)
    h1 = _dense(agg1, w, x, _permute_w(W1), b1.reshape(1, D))

    agg2 = _sc_agg(_pack_table(h1), lists)
    h2 = _dense(agg2, w, h1, _permute_w(W2), b2.reshape(1, D))
    return h2


# speedup vs baseline: 1.1962x; 1.0058x over previous
import dataclasses

import jax
import jax.numpy as jnp
from jax import lax
from jax.experimental import pallas as pl
from jax.experimental.pallas import tpu as pltpu
from jax.experimental.pallas import tpu_sc as plsc

N_NODES = 10000
N_EDGES = 160000
D = 256
NC = 2
NS = 16
NW = NC * NS
L = 16
HN = 5120
AP = 5248
AP2 = 2 * AP
R_PER_SUB2 = AP2 // NS
NP = 10240
K1 = 128
E_SCAN = 5120
NCH1 = E_SCAN // K1
SB = 32
MAXCH = 42
DROP = 1 << 20
PAD_LOCAL = HN + 32
R_TC = 1024

_SC_COMPILER_PARAMS = pltpu.CompilerParams()
if "needs_layout_passes" in pltpu.CompilerParams.__dataclass_fields__:
    _SC_COMPILER_PARAMS = dataclasses.replace(
        _SC_COMPILER_PARAMS, needs_layout_passes=False)

_PERM = []
for _p in range(D):
    _m, _r = divmod(_p, 32)
    _PERM.append(32 * _m + (2 * _r if _r < 16 else 2 * (_r - 16) + 1))


def _make_partition():
    mesh = plsc.VectorSubcoreMesh(core_axis_name="c", subcore_axis_name="s")

    def body(epk, lists, out_w, stage, pb0, pb1, w_part, kbuf, cbuf):
        c = lax.axis_index("c")
        s = lax.axis_index("s")
        w = c * NS + s
        iota = lax.iota(jnp.int32, L)
        planes = (pb0, pb1)

        kbuf[pl.ds(L, L)] = jnp.full((L,), -1, jnp.int32)

        @pl.loop(0, NP // L)
        def _(r):
            w_part[pl.ds(r * L, L)] = jnp.zeros((L,), jnp.float32)

        def scatter_triple(pb, idxv, sv, dv, cv, m):
            ch = lax.shift_right_logical(idxv, 7)
            ln = lax.bitwise_and(idxv, 127)
            plsc.store_scatter(pb, [ch, jnp.zeros((L,), jnp.int32), ln],
                               sv, mask=m)
            plsc.store_scatter(pb, [ch, jnp.ones((L,), jnp.int32), ln],
                               dv, mask=m)
            plsc.store_scatter(pb, [ch, jnp.full((L,), 2, jnp.int32), ln],
                               cv, mask=m)

        def chunk(k, curs):
            pltpu.sync_copy(epk.at[w, k], stage)
            cur0, cur1 = curs
            for g in range(K1 // L):
                sl = pl.ds(g * L, L)
                d = stage[1, sl]
                sv = stage[0, sl]
                cv = stage[2, sl]
                new = []
                for h, cur in ((0, cur0), (1, cur1)):
                    m = (d >= h * HN) & (d < (h + 1) * HN)
                    ones = jnp.where(m, 1, 0)
                    pc = plsc.cumsum(ones)
                    idxv = pc - 1 + cur
                    scatter_triple(planes[h], idxv, sv, d - h * HN, cv, m)
                    new.append(cur + jnp.sum(ones))
                cur0, cur1 = new

                vf = plsc.bitcast(cv, jnp.float32)
                ds_, vs_ = plsc.sort_key_val(d, vf)
                kbuf[pl.ds(0, L)] = ds_
                knext = plsc.load_gather(kbuf, [iota + 1])
                is_last = (ds_ != knext) & (ds_ < N_NODES)
                cum = plsc.cumsum(vs_)
                cbuf[pl.ds(0, L)] = cum
                kprev = plsc.load_gather(kbuf, [jnp.maximum(iota - 1, 0)])
                is_first = (ds_ != kprev) | (iota == 0)
                start = plsc.cummax(jnp.where(is_first, iota, 0))
                pc2 = plsc.load_gather(cbuf, [jnp.maximum(start - 1, 0)])
                prev = jnp.where(start == 0, 0.0, pc2)
                plsc.addupdate_scatter(w_part, [ds_], cum - prev,
                                       mask=is_last)
            return cur0, cur1

        cur0, cur1 = lax.fori_loop(0, NCH1, chunk,
                                   (jnp.int32(0), jnp.int32(0)))

        for h, cur in ((0, cur0), (1, cur1)):
            for t in range(K1 // L):
                idxv = cur + t * L + iota
                scatter_triple(planes[h], idxv,
                               jnp.zeros((L,), jnp.int32),
                               jnp.full((L,), PAD_LOCAL, jnp.int32),
                               jnp.zeros((L,), jnp.int32), None)
            nch = (cur + K1 - 1) // K1
            plsc.store_scatter(planes[h],
                               [jnp.zeros((L,), jnp.int32),
                                jnp.full((L,), 3, jnp.int32), iota],
                               jnp.where(iota == 0, nch, 0))
            pltpu.sync_copy(planes[h], lists.at[w, h])

        pltpu.sync_copy(w_part, out_w.at[w])

    return pl.kernel(
        body,
        out_type=[
            jax.ShapeDtypeStruct((NW, NC, MAXCH, 4, K1), jnp.int32),
            jax.ShapeDtypeStruct((NW, NP), jnp.float32),
        ],
        mesh=mesh,
        scratch_types=[
            pltpu.VMEM((3, K1), jnp.int32),
            pltpu.VMEM((MAXCH, 4, K1), jnp.int32),
            pltpu.VMEM((MAXCH, 4, K1), jnp.int32),
            pltpu.VMEM((NP,), jnp.float32),
            pltpu.VMEM((2 * L,), jnp.int32),
            pltpu.VMEM((L,), jnp.float32),
        ],
        compiler_params=_SC_COMPILER_PARAMS,
    )


def _make_sc_aggregate():
    mesh = plsc.VectorSubcoreMesh(core_axis_name="c", subcore_axis_name="s")

    def body(tab, lists, out, acc, sts0, sts1, dstc_a, dstc_b,
             rows_i0, rows_i1, rows_fa, rows_fb,
             isem0, isem1, gsem0, gsem1, ssem_a, ssem_b):
        sts = (sts0, sts1)
        rows_i = (rows_i0, rows_i1)
        isem = (isem0, isem1)
        gsem = (gsem0, gsem1)
        c = lax.axis_index("c")
        s = lax.axis_index("s")
        iota = lax.iota(jnp.int32, L)

        @pl.loop(0, K1)
        def _(r):
            for j in range(K1 // L):
                rows_fa[r, pl.ds(j * L, L)] = jnp.zeros((L,), jnp.float32)

        base = s * R_PER_SUB2
        for i in range(5):
            pltpu.sync_copy(rows_fa, acc.at[pl.ds(base + i * K1, K1)])
        pltpu.sync_copy(rows_fa.at[pl.ds(0, 16)],
                        acc.at[pl.ds(base + 5 * K1, 16)])
        plsc.subcore_barrier()

        def stage_idx(wsel, j, b):
            pltpu.async_copy(lists.at[wsel, c, j], sts[b], isem[b])

        def wait_idx(b):
            pltpu.make_async_copy(lists.at[0, 0, 0], sts[b], isem[b]).wait()

        def fire_gather(b, q, p):
            pltpu.async_copy(tab.at[sts[b].at[0, pl.ds(q * SB, SB)]],
                             rows_i[p], gsem[p])

        def wait_gather(b, q, p):
            pltpu.make_async_copy(tab.at[sts[b].at[0, pl.ds(q * SB, SB)]],
                                  rows_i[p], gsem[p]).wait()

        def fire_scatter():
            pltpu.async_copy(rows_fa, acc.at[dstc_a.at[0]], ssem_a,
                             add=True)
            pltpu.async_copy(rows_fb, acc.at[dstc_b.at[0]], ssem_b,
                             add=True)

        def wait_scatter():
            pltpu.make_async_copy(rows_fa, acc.at[dstc_a.at[0]],
                                  ssem_a).wait()
            pltpu.make_async_copy(rows_fb, acc.at[dstc_b.at[0]],
                                  ssem_b).wait()

        def scale_sub(b, q, p):
            @plsc.parallel_loop(0, SB, step=4, unroll=2)
            def _(e):
                for u in range(4):
                    ee = e + u
                    ce = plsc.bitcast(
                        plsc.load_gather(
                            sts[b],
                            [jnp.full((L,), 2, jnp.int32),
                             jnp.full((L,), q * SB + ee, jnp.int32)]),
                        jnp.float32)
                    for m in range(D // 32):
                        iv = rows_i[p][ee, pl.ds(m * L, L)]
                        bv = plsc.bitcast(iv, jnp.bfloat16)
                        av, bv2 = plsc.unpack(
                            bv, format=plsc.PackFormat.INTERLEAVED)
                        tgt = rows_fa if m < 4 else rows_fb
                        mm = m % 4
                        tgt[q * SB + ee, pl.ds(mm * 32, L)] = av * ce
                        tgt[q * SB + ee, pl.ds(mm * 32 + L, L)] = bv2 * ce

        def consume(j, b, n):
            fire_gather(b, 1, 1)

            @pl.when(j >= 1)
            def _():
                wait_scatter()

            wait_gather(b, 0, 0)
            scale_sub(b, 0, 0)
            fire_gather(b, 2, 0)
            wait_gather(b, 1, 1)
            scale_sub(b, 1, 1)
            fire_gather(b, 3, 1)
            wait_gather(b, 2, 0)
            scale_sub(b, 2, 0)

            @pl.when(j + 1 < n)
            def _():
                if b == 0:
                    wait_idx(1)
                    fire_gather(1, 0, 0)
                else:
                    wait_idx(0)
                    fire_gather(0, 0, 0)

            wait_gather(b, 3, 1)
            scale_sub(b, 3, 1)

            for g in range(K1 // L):
                sl = pl.ds(g * L, L)
                dloc2 = sts[b][1, sl] * 2
                dstc_a[0, sl] = dloc2
                dstc_b[0, sl] = dloc2 + 1
            fire_scatter()

        def run_list(wsel):
            stage_idx(wsel, jnp.int32(0), 0)
            wait_idx(0)
            cv = sts0[3, pl.ds(0, L)]
            n = jnp.sum(jnp.where(iota == 0, cv, 0))
            stage_idx(wsel, jnp.int32(1), 1)

            @pl.when(n > 0)
            def _():
                fire_gather(0, 0, 0)

            @pl.loop(0, n)
            def _(j):
                b = lax.rem(j, 2)

                @pl.when(b == 0)
                def _():
                    consume(j, 0, n)

                    @pl.when(j + 2 < n)
                    def _():
                        stage_idx(wsel, j + 2, 0)

                @pl.when(b == 1)
                def _():
                    consume(j, 1, n)

                    @pl.when(j + 2 < n)
                    def _():
                        stage_idx(wsel, j + 2, 1)

            @pl.when(n >= 1)
            def _():
                wait_scatter()

            @pl.when(n <= 1)
            def _():
                wait_idx(1)

        run_list(2 * s)
        run_list(2 * s + 1)

        plsc.subcore_barrier()
        for i in range(5):
            pltpu.sync_copy(acc.at[pl.ds(base + i * K1, K1)], rows_fa)
            pltpu.sync_copy(rows_fa, out.at[c, pl.ds(base + i * K1, K1)])
        pltpu.sync_copy(acc.at[pl.ds(base + 5 * K1, 16)],
                        rows_fa.at[pl.ds(0, 16)])
        pltpu.sync_copy(rows_fa.at[pl.ds(0, 16)],
                        out.at[c, pl.ds(base + 5 * K1, 16)])

    return pl.kernel(
        body,
        out_type=jax.ShapeDtypeStruct((NC, AP2, K1), jnp.float32),
        mesh=mesh,
        scratch_types=[
            pltpu.VMEM_SHARED((AP2, K1), jnp.float32),
            pltpu.VMEM((4, K1), jnp.int32),
            pltpu.VMEM((4, K1), jnp.int32),
            pltpu.VMEM((1, K1), jnp.int32),
            pltpu.VMEM((1, K1), jnp.int32),
            pltpu.VMEM((SB, D // 2), jnp.int32),
            pltpu.VMEM((SB, D // 2), jnp.int32),
            pltpu.VMEM((K1, K1), jnp.float32),
            pltpu.VMEM((K1, K1), jnp.float32),
            pltpu.SemaphoreType.DMA,
            pltpu.SemaphoreType.DMA,
            pltpu.SemaphoreType.DMA,
            pltpu.SemaphoreType.DMA,
            pltpu.SemaphoreType.DMA,
            pltpu.SemaphoreType.DMA,
        ],
        compiler_params=_SC_COMPILER_PARAMS,
    )


def _tc_pre(h, wb, b):
    def body(h_ref, wb_ref, b_ref, o_ref):
        o_ref[...] = jnp.dot(h_ref[...], wb_ref[...],
                             preferred_element_type=jnp.float32) + b_ref[0]

    f = pl.pallas_call(
        body,
        grid=(N_NODES // R_TC + 1,),
        in_specs=[
            pl.BlockSpec((R_TC, D), lambda i: (i, 0)),
            pl.BlockSpec((D, D), lambda i: (0, 0)),
            pl.BlockSpec((1, D), lambda i: (0, 0)),
        ],
        out_specs=pl.BlockSpec((R_TC, D), lambda i: (i, 0)),
        out_shape=jax.ShapeDtypeStruct((N_NODES, D), jnp.float32),
    )
    return f(h, wb, b)


def _tc_combine(g, w, pre, wt):
    def body(g_ref, w_ref, p_ref, wt_ref, o_ref):
        wsum = jnp.sum(w_ref[...], axis=0)
        inv = 1.0 / jnp.maximum(wsum, 1.0)
        n = jnp.reshape(g_ref[0], (R_TC, D)) * inv[:, None]
        z = jnp.dot(n, wt_ref[...],
                    preferred_element_type=jnp.float32) + p_ref[...]
        z = jnp.maximum(z, 0.0)
        nrm = jnp.sqrt(jnp.sum(z * z, axis=1, keepdims=True))
        nrm = jnp.where(nrm == 0.0, 1.0, nrm)
        o_ref[...] = z / nrm

    nhb = HN // R_TC
    f = pl.pallas_call(
        body,
        grid=(NC * nhb,),
        in_specs=[
            pl.BlockSpec((1, 2 * R_TC, K1), lambda i: (i // 5, i % 5, 0)),
            pl.BlockSpec((NW, R_TC), lambda i: (0, i)),
            pl.BlockSpec((R_TC, D), lambda i: (i, 0)),
            pl.BlockSpec((D, D), lambda i: (0, 0)),
        ],
        out_specs=pl.BlockSpec((R_TC, D), lambda i: (i, 0)),
        out_shape=jax.ShapeDtypeStruct((N_NODES, D), jnp.float32),
    )
    return f(g, w, pre, wt)


_partition = _make_partition()
_sc_agg = _make_sc_aggregate()


def _pack_table(h):
    hb = h.astype(jnp.bfloat16)
    return lax.bitcast_convert_type(hb.reshape(N_NODES, D // 2, 2),
                                    jnp.int32)


def _permute_wt(wmat):
    perm = jnp.array(_PERM, dtype=jnp.int32)
    return wmat[:D][perm]


def kernel(x, edge_index, edge_count, W1, b1, W2, b2):
    epr = N_EDGES // NW
    src = edge_index[0].astype(jnp.int32).reshape(NW, epr)
    dst = edge_index[1].astype(jnp.int32).reshape(NW, epr)
    cnt = edge_count.astype(jnp.float32).reshape(NW, epr)
    pad = E_SCAN - epr
    src = jnp.pad(src, ((0, 0), (0, pad)))
    dst = jnp.pad(dst, ((0, 0), (0, pad)), constant_values=DROP)
    cnt_bits = lax.bitcast_convert_type(
        jnp.pad(cnt, ((0, 0), (0, pad))), jnp.int32)
    epk = jnp.stack([src.reshape(NW, NCH1, K1),
                     dst.reshape(NW, NCH1, K1),
                     cnt_bits.reshape(NW, NCH1, K1)], axis=2)

    lists, w = _partition(epk)

    agg1 = _sc_agg(_pack_table(x), lists)
    pre1 = _tc_pre(x, W1[D:], b1.reshape(1, D))
    h1 = _tc_combine(agg1, w, pre1, _permute_wt(W1))

    agg2 = _sc_agg(_pack_table(h1), lists)
    pre2 = _tc_pre(h1, W2[D:], b2.reshape(1, D))
    h2 = _tc_combine(agg2, w, pre2, _permute_wt(W2))
    return h2


# speedup vs baseline: 1.4708x; 1.2296x over previous
import dataclasses

import jax
import jax.numpy as jnp
from jax import lax
from jax.experimental import pallas as pl
from jax.experimental.pallas import tpu as pltpu
from jax.experimental.pallas import tpu_sc as plsc

N_NODES = 10000
N_EDGES = 160000
D = 256
DH = 128
NC = 2
NS = 16
L = 16
K = 128
E_PER_SUB = 10240
NCHUNK = E_PER_SUB // K
PAD_E = E_PER_SUB - N_EDGES // NS
NP = 10240
R_PER_SUB = NP // NS
ZR = 128
PAD_DST = 10200
R_TC = 1024

_SC_COMPILER_PARAMS = pltpu.CompilerParams()
if "needs_layout_passes" in pltpu.CompilerParams.__dataclass_fields__:
    _SC_COMPILER_PARAMS = dataclasses.replace(
        _SC_COMPILER_PARAMS, needs_layout_passes=False)


def _make_sc_aggregate(need_w):
    mesh = plsc.VectorSubcoreMesh(core_axis_name="c", subcore_axis_name="s")

    def body(tab, ed3, *refs):
        if need_w:
            (out, out_w, acc, idx0, idx1, dstc0, dstc1, rows0, rows1,
             w_part, kbuf, cbuf,
             isem0, isem1, gsem0, gsem1, ssem0, ssem1) = refs
        else:
            (out, acc, idx0, idx1, dstc0, dstc1, rows0, rows1,
             isem0, isem1, gsem0, gsem1, ssem0, ssem1) = refs
        idx = (idx0, idx1)
        dstc = (dstc0, dstc1)
        rows = (rows0, rows1)
        isem = (isem0, isem1)
        gsem = (gsem0, gsem1)
        ssem = (ssem0, ssem1)
        c = lax.axis_index("c")
        s = lax.axis_index("s")
        iota = lax.iota(jnp.int32, L)
        off = jnp.full((L,), c * N_NODES, jnp.int32)

        @pl.loop(0, ZR)
        def _(r):
            for j in range(DH // L):
                rows0[r, pl.ds(j * L, L)] = jnp.zeros((L,), jnp.float32)

        for i in range(R_PER_SUB // ZR):
            pltpu.sync_copy(rows0, acc.at[pl.ds(s * R_PER_SUB + i * ZR, ZR)])

        if need_w:
            @pl.when(c == 0)
            def _():
                kbuf[pl.ds(L, L)] = jnp.full((L,), -1, jnp.int32)

                @pl.loop(0, NP // L)
                def _(r):
                    w_part[pl.ds(r * L, L)] = jnp.zeros((L,), jnp.float32)

        plsc.subcore_barrier()

        def stage_idx(k, b):
            pltpu.async_copy(ed3.at[s, k], idx[b], isem[b])

        def wait_idx(b):
            pltpu.make_async_copy(ed3.at[s, 0], idx[b], isem[b]).wait()

        def fire_gather(b):
            for g in range(K // L):
                sl = pl.ds(g * L, L)
                idx[b][0, sl] = idx[b][0, sl] + off
            pltpu.async_copy(tab.at[idx[b].at[0]], rows[b], gsem[b])

        def wait_gather(b):
            pltpu.make_async_copy(tab.at[idx[b].at[0]], rows[b],
                                  gsem[b]).wait()

        def consume(b):
            for g in range(K // L):
                sl = pl.ds(g * L, L)
                dstc[b][0, sl] = idx[b][1, sl]

            @plsc.parallel_loop(0, K, step=4, unroll=2)
            def _(e):
                for u in range(4):
                    ce = plsc.bitcast(
                        plsc.load_gather(
                            idx[b], [jnp.full((L,), 2, jnp.int32),
                                     jnp.full((L,), e + u, jnp.int32)]),
                        jnp.float32)
                    for j in range(DH // L):
                        sl = pl.ds(j * L, L)
                        rows[b][e + u, sl] = rows[b][e + u, sl] * ce

            if need_w:
                @pl.when(c == 0)
                def _():
                    for g in range(K // L):
                        sl = pl.ds(g * L, L)
                        d = idx[b][1, sl]
                        v = plsc.bitcast(idx[b][2, sl], jnp.float32)
                        ds_, vs_ = plsc.sort_key_val(d, v)
                        kbuf[pl.ds(0, L)] = ds_
                        knext = plsc.load_gather(kbuf, [iota + 1])
                        is_last = ds_ != knext
                        cum = plsc.cumsum(vs_)
                        cbuf[pl.ds(0, L)] = cum
                        kprev = plsc.load_gather(
                            kbuf, [jnp.maximum(iota - 1, 0)])
                        is_first = (ds_ != kprev) | (iota == 0)
                        start = plsc.cummax(jnp.where(is_first, iota, 0))
                        pc = plsc.load_gather(
                            cbuf, [jnp.maximum(start - 1, 0)])
                        prev = jnp.where(start == 0, 0.0, pc)
                        plsc.addupdate_scatter(
                            w_part, [ds_], cum - prev, mask=is_last)

        def fire_scatter(b):
            pltpu.async_copy(rows[b], acc.at[dstc[b].at[0]], ssem[b],
                             add=True)

        def wait_scatter(b):
            pltpu.make_async_copy(rows[b], acc.at[dstc[b].at[0]],
                                  ssem[b]).wait()

        stage_idx(0, 0)
        stage_idx(1, 1)
        wait_idx(0)
        fire_gather(0)

        @pl.loop(0, NCHUNK)
        def _(k):
            b = lax.rem(k, 2)

            @pl.when(k < NCHUNK - 1)
            def _():
                @pl.when(b == 0)
                def _():
                    wait_idx(1)

                @pl.when(b == 1)
                def _():
                    wait_idx(0)

            @pl.when(k >= 1)
            def _():
                @pl.when(b == 0)
                def _():
                    wait_scatter(1)

                @pl.when(b == 1)
                def _():
                    wait_scatter(0)

            @pl.when(k < NCHUNK - 1)
            def _():
                @pl.when(b == 0)
                def _():
                    fire_gather(1)

                @pl.when(b == 1)
                def _():
                    fire_gather(0)

            @pl.when(b == 0)
            def _():
                wait_gather(0)
                consume(0)
                fire_scatter(0)

                @pl.when(k < NCHUNK - 2)
                def _():
                    stage_idx(k + 2, 0)

            @pl.when(b == 1)
            def _():
                wait_gather(1)
                consume(1)
                fire_scatter(1)

                @pl.when(k < NCHUNK - 2)
                def _():
                    stage_idx(k + 2, 1)

        wait_scatter((NCHUNK - 1) % 2)
        plsc.subcore_barrier()
        for i in range(R_PER_SUB // ZR):
            r0 = s * R_PER_SUB + i * ZR
            pltpu.sync_copy(acc.at[pl.ds(r0, ZR)], rows0)
            pltpu.sync_copy(rows0, out.at[c, pl.ds(r0, ZR)])
        if need_w:
            @pl.when(c == 0)
            def _():
                pltpu.sync_copy(w_part, out_w.at[s])

    out_type = [jax.ShapeDtypeStruct((NC, NP, DH), jnp.float32)]
    if need_w:
        out_type.append(jax.ShapeDtypeStruct((NS, NP), jnp.float32))
    scratch = [
        pltpu.VMEM_SHARED((NP, DH), jnp.float32),
        pltpu.VMEM((3, K), jnp.int32),
        pltpu.VMEM((3, K), jnp.int32),
        pltpu.VMEM((1, K), jnp.int32),
        pltpu.VMEM((1, K), jnp.int32),
        pltpu.VMEM((K, DH), jnp.float32),
        pltpu.VMEM((K, DH), jnp.float32),
    ]
    if need_w:
        scratch += [
            pltpu.VMEM((NP,), jnp.float32),
            pltpu.VMEM((2 * L,), jnp.int32),
            pltpu.VMEM((L,), jnp.float32),
        ]
    scratch += [pltpu.SemaphoreType.DMA] * 6

    return pl.kernel(
        body,
        out_type=out_type,
        mesh=mesh,
        scratch_types=scratch,
        compiler_params=_SC_COMPILER_PARAMS,
    )


def _dense(g, w, h, wm, b):
    def body(g0_ref, g1_ref, w_ref, h_ref, wm_ref, b_ref, o_ref):
        wsum = jnp.sum(w_ref[...], axis=0)
        inv = 1.0 / jnp.maximum(wsum, 1.0)
        n0 = g0_ref[0] * inv[:, None]
        n1 = g1_ref[0] * inv[:, None]
        wmat = wm_ref[...]
        z = (jnp.dot(n0, wmat[:DH], preferred_element_type=jnp.float32)
             + jnp.dot(n1, wmat[DH:2 * DH], preferred_element_type=jnp.float32)
             + jnp.dot(h_ref[...], wmat[2 * DH:],
                       preferred_element_type=jnp.float32))
        z = jnp.maximum(z + b_ref[0], 0.0)
        nrm = jnp.sqrt(jnp.sum(z * z, axis=1, keepdims=True))
        nrm = jnp.where(nrm == 0.0, 1.0, nrm)
        o_ref[...] = z / nrm

    nb = NP // R_TC
    f = pl.pallas_call(
        body,
        grid=(nb,),
        in_specs=[
            pl.BlockSpec((1, R_TC, DH), lambda i: (0, i, 0)),
            pl.BlockSpec((1, R_TC, DH), lambda i: (1, i, 0)),
            pl.BlockSpec((NS, R_TC), lambda i: (0, i)),
            pl.BlockSpec((R_TC, D), lambda i: (i, 0)),
            pl.BlockSpec((2 * D, D), lambda i: (0, 0)),
            pl.BlockSpec((1, D), lambda i: (0, 0)),
        ],
        out_specs=pl.BlockSpec((R_TC, D), lambda i: (i, 0)),
        out_shape=jax.ShapeDtypeStruct((N_NODES, D), jnp.float32),
    )
    return f(g, g, w, h, wm, b)


_sc_agg_w = _make_sc_aggregate(True)
_sc_agg = _make_sc_aggregate(False)


def _pack_edges(edge_index, edge_count):
    epr = N_EDGES // NS
    src = edge_index[0].astype(jnp.int32).reshape(NS, epr)
    dst = edge_index[1].astype(jnp.int32).reshape(NS, epr)
    cnt = edge_count.astype(jnp.float32).reshape(NS, epr)
    src = jnp.pad(src, ((0, 0), (0, PAD_E)))
    dst = jnp.pad(dst, ((0, 0), (0, PAD_E)), constant_values=PAD_DST)
    cnt = jnp.pad(cnt, ((0, 0), (0, PAD_E)))
    cnt_bits = lax.bitcast_convert_type(cnt, jnp.int32)
    ed3 = jnp.stack([src.reshape(NS, NCHUNK, K),
                     dst.reshape(NS, NCHUNK, K),
                     cnt_bits.reshape(NS, NCHUNK, K)], axis=2)
    return ed3


def kernel(x, edge_index, edge_count, W1, b1, W2, b2):
    ed3 = _pack_edges(edge_index, edge_count)

    tab1 = jnp.concatenate([x[:, :DH], x[:, DH:]], axis=0)
    agg1, w = _sc_agg_w(tab1, ed3)
    h1 = _dense(agg1, w, x, W1, b1.reshape(1, D))

    tab2 = jnp.concatenate([h1[:, :DH], h1[:, DH:]], axis=0)
    (agg2,) = _sc_agg(tab2, ed3)
    h2 = _dense(agg2, w, h1, W2, b2.reshape(1, D))
    return h2


# speedup vs baseline: 1.4725x; 1.0011x over previous
import dataclasses

import jax
import jax.numpy as jnp
from jax import lax
from jax.experimental import pallas as pl
from jax.experimental.pallas import tpu as pltpu
from jax.experimental.pallas import tpu_sc as plsc

N_NODES = 10000
N_EDGES = 160000
D = 256
DH = 128
NC = 2
NS = 16
L = 16
K = 128
E_PER_SUB = 10240
NCHUNK = E_PER_SUB // K
PAD_E = E_PER_SUB - N_EDGES // NS
NP = 10240
R_PER_SUB = NP // NS
ZR = 128
PAD_DST = 10200
R_TC = 1024

_SC_COMPILER_PARAMS = pltpu.CompilerParams()
if "needs_layout_passes" in pltpu.CompilerParams.__dataclass_fields__:
    _SC_COMPILER_PARAMS = dataclasses.replace(
        _SC_COMPILER_PARAMS, needs_layout_passes=False)


def _make_sc_aggregate(need_w):
    mesh = plsc.VectorSubcoreMesh(core_axis_name="c", subcore_axis_name="s")

    def body(tab, ed3, *refs):
        if need_w:
            (out, out_w, acc, idx0, idx1, dstc0, dstc1, rows0, rows1,
             w_part, kbuf, cbuf,
             isem0, isem1, gsem0, gsem1, ssem0, ssem1) = refs
        else:
            (out, acc, idx0, idx1, dstc0, dstc1, rows0, rows1,
             isem0, isem1, gsem0, gsem1, ssem0, ssem1) = refs
        idx = (idx0, idx1)
        dstc = (dstc0, dstc1)
        rows = (rows0, rows1)
        isem = (isem0, isem1)
        gsem = (gsem0, gsem1)
        ssem = (ssem0, ssem1)
        c = lax.axis_index("c")
        s = lax.axis_index("s")
        iota = lax.iota(jnp.int32, L)
        off = jnp.full((L,), c * N_NODES, jnp.int32)

        @pl.loop(0, ZR)
        def _(r):
            for j in range(DH // L):
                rows0[r, pl.ds(j * L, L)] = jnp.zeros((L,), jnp.float32)

        for i in range(R_PER_SUB // ZR):
            pltpu.sync_copy(rows0, acc.at[pl.ds(s * R_PER_SUB + i * ZR, ZR)])

        if need_w:
            @pl.when(c == 0)
            def _():
                kbuf[pl.ds(L, L)] = jnp.full((L,), -1, jnp.int32)

                @pl.loop(0, NP // L)
                def _(r):
                    w_part[pl.ds(r * L, L)] = jnp.zeros((L,), jnp.float32)

        plsc.subcore_barrier()

        def stage_idx(k, b):
            pltpu.async_copy(ed3.at[s, k], idx[b], isem[b])

        def wait_idx(b):
            pltpu.make_async_copy(ed3.at[s, 0], idx[b], isem[b]).wait()

        def fire_gather(b):
            for g in range(K // L):
                sl = pl.ds(g * L, L)
                idx[b][0, sl] = idx[b][0, sl] + off
            pltpu.async_copy(tab.at[idx[b].at[0]], rows[b], gsem[b])

        def wait_gather(b):
            pltpu.make_async_copy(tab.at[idx[b].at[0]], rows[b],
                                  gsem[b]).wait()

        def consume(b):
            for g in range(K // L):
                sl = pl.ds(g * L, L)
                dstc[b][0, sl] = idx[b][1, sl]

            @plsc.parallel_loop(0, K, step=4, unroll=2)
            def _(e):
                for u in range(4):
                    ce = plsc.bitcast(
                        plsc.load_gather(
                            idx[b], [jnp.full((L,), 2, jnp.int32),
                                     jnp.full((L,), e + u, jnp.int32)]),
                        jnp.float32)
                    for j in range(DH // L):
                        sl = pl.ds(j * L, L)
                        rows[b][e + u, sl] = rows[b][e + u, sl] * ce

            if need_w:
                @pl.when(c == 0)
                def _():
                    for g in range(K // L):
                        sl = pl.ds(g * L, L)
                        d = idx[b][1, sl]
                        v = plsc.bitcast(idx[b][2, sl], jnp.float32)
                        ds_, vs_ = plsc.sort_key_val(d, v)
                        kbuf[pl.ds(0, L)] = ds_
                        knext = plsc.load_gather(kbuf, [iota + 1])
                        is_last = ds_ != knext
                        cum = plsc.cumsum(vs_)
                        cbuf[pl.ds(0, L)] = cum
                        kprev = plsc.load_gather(
                            kbuf, [jnp.maximum(iota - 1, 0)])
                        is_first = (ds_ != kprev) | (iota == 0)
                        start = plsc.cummax(jnp.where(is_first, iota, 0))
                        pc = plsc.load_gather(
                            cbuf, [jnp.maximum(start - 1, 0)])
                        prev = jnp.where(start == 0, 0.0, pc)
                        plsc.addupdate_scatter(
                            w_part, [ds_], cum - prev, mask=is_last)

        def fire_scatter(b):
            pltpu.async_copy(rows[b], acc.at[dstc[b].at[0]], ssem[b],
                             add=True)

        def wait_scatter(b):
            pltpu.make_async_copy(rows[b], acc.at[dstc[b].at[0]],
                                  ssem[b]).wait()

        stage_idx(0, 0)
        stage_idx(1, 1)
        wait_idx(0)
        fire_gather(0)

        @pl.loop(0, NCHUNK)
        def _(k):
            b = lax.rem(k, 2)

            @pl.when(k < NCHUNK - 1)
            def _():
                @pl.when(b == 0)
                def _():
                    wait_idx(1)

                @pl.when(b == 1)
                def _():
                    wait_idx(0)

            @pl.when(k >= 1)
            def _():
                @pl.when(b == 0)
                def _():
                    wait_scatter(1)

                @pl.when(b == 1)
                def _():
                    wait_scatter(0)

            @pl.when(k < NCHUNK - 1)
            def _():
                @pl.when(b == 0)
                def _():
                    fire_gather(1)

                @pl.when(b == 1)
                def _():
                    fire_gather(0)

            @pl.when(b == 0)
            def _():
                wait_gather(0)
                consume(0)
                fire_scatter(0)

                @pl.when(k < NCHUNK - 2)
                def _():
                    stage_idx(k + 2, 0)

            @pl.when(b == 1)
            def _():
                wait_gather(1)
                consume(1)
                fire_scatter(1)

                @pl.when(k < NCHUNK - 2)
                def _():
                    stage_idx(k + 2, 1)

        wait_scatter((NCHUNK - 1) % 2)
        plsc.subcore_barrier()
        for i in range(R_PER_SUB // ZR):
            r0 = s * R_PER_SUB + i * ZR
            pltpu.sync_copy(acc.at[pl.ds(r0, ZR)], rows0)
            pltpu.sync_copy(rows0, out.at[c, pl.ds(r0, ZR)])
        if need_w:
            @pl.when(c == 0)
            def _():
                pltpu.sync_copy(w_part, out_w.at[s])

    out_type = [jax.ShapeDtypeStruct((NC, NP, DH), jnp.float32)]
    if need_w:
        out_type.append(jax.ShapeDtypeStruct((NS, NP), jnp.float32))
    scratch = [
        pltpu.VMEM_SHARED((NP, DH), jnp.float32),
        pltpu.VMEM((3, K), jnp.int32),
        pltpu.VMEM((3, K), jnp.int32),
        pltpu.VMEM((1, K), jnp.int32),
        pltpu.VMEM((1, K), jnp.int32),
        pltpu.VMEM((K, DH), jnp.float32),
        pltpu.VMEM((K, DH), jnp.float32),
    ]
    if need_w:
        scratch += [
            pltpu.VMEM((NP,), jnp.float32),
            pltpu.VMEM((2 * L,), jnp.int32),
            pltpu.VMEM((L,), jnp.float32),
        ]
    scratch += [pltpu.SemaphoreType.DMA] * 6

    return pl.kernel(
        body,
        out_type=out_type,
        mesh=mesh,
        scratch_types=scratch,
        compiler_params=_SC_COMPILER_PARAMS,
    )


def _tc_pre(h, wb, b):
    def body(h_ref, wb_ref, b_ref, o_ref):
        o_ref[...] = jnp.dot(h_ref[...], wb_ref[...],
                             preferred_element_type=jnp.float32) + b_ref[0]

    f = pl.pallas_call(
        body,
        grid=(N_NODES // R_TC,),
        in_specs=[
            pl.BlockSpec((R_TC, D), lambda i: (i, 0)),
            pl.BlockSpec((D, D), lambda i: (0, 0)),
            pl.BlockSpec((1, D), lambda i: (0, 0)),
        ],
        out_specs=pl.BlockSpec((R_TC, D), lambda i: (i, 0)),
        out_shape=jax.ShapeDtypeStruct((N_NODES, D), jnp.float32),
    )
    return f(h, wb, b)


def _tc_combine(g, w, pre, wt):
    def body(g0_ref, g1_ref, w_ref, p_ref, wt_ref, o_ref):
        wsum = jnp.sum(w_ref[...], axis=0)
        inv = 1.0 / jnp.maximum(wsum, 1.0)
        n0 = g0_ref[0] * inv[:, None]
        n1 = g1_ref[0] * inv[:, None]
        wtm = wt_ref[...]
        z = (jnp.dot(n0, wtm[:DH], preferred_element_type=jnp.float32)
             + jnp.dot(n1, wtm[DH:], preferred_element_type=jnp.float32)
             + p_ref[...])
        z = jnp.maximum(z, 0.0)
        nrm = jnp.sqrt(jnp.sum(z * z, axis=1, keepdims=True))
        nrm = jnp.where(nrm == 0.0, 1.0, nrm)
        o_ref[...] = z / nrm

    f = pl.pallas_call(
        body,
        grid=(NP // R_TC,),
        in_specs=[
            pl.BlockSpec((1, R_TC, DH), lambda i: (0, i, 0)),
            pl.BlockSpec((1, R_TC, DH), lambda i: (1, i, 0)),
            pl.BlockSpec((NS, R_TC), lambda i: (0, i)),
            pl.BlockSpec((R_TC, D), lambda i: (i, 0)),
            pl.BlockSpec((D, D), lambda i: (0, 0)),
        ],
        out_specs=pl.BlockSpec((R_TC, D), lambda i: (i, 0)),
        out_shape=jax.ShapeDtypeStruct((N_NODES, D), jnp.float32),
    )
    return f(g, g, w, pre, wt)


_sc_agg_w = _make_sc_aggregate(True)
_sc_agg = _make_sc_aggregate(False)


def _pack_edges(edge_index, edge_count):
    epr = N_EDGES // NS
    src = edge_index[0].astype(jnp.int32).reshape(NS, epr)
    dst = edge_index[1].astype(jnp.int32).reshape(NS, epr)
    cnt = edge_count.astype(jnp.float32).reshape(NS, epr)
    src = jnp.pad(src, ((0, 0), (0, PAD_E)))
    dst = jnp.pad(dst, ((0, 0), (0, PAD_E)), constant_values=PAD_DST)
    cnt = jnp.pad(cnt, ((0, 0), (0, PAD_E)))
    cnt_bits = lax.bitcast_convert_type(cnt, jnp.int32)
    ed3 = jnp.stack([src.reshape(NS, NCHUNK, K),
                     dst.reshape(NS, NCHUNK, K),
                     cnt_bits.reshape(NS, NCHUNK, K)], axis=2)
    return ed3


def kernel(x, edge_index, edge_count, W1, b1, W2, b2):
    ed3 = _pack_edges(edge_index, edge_count)

    tab1 = jnp.concatenate([x[:, :DH], x[:, DH:]], axis=0)
    agg1, w = _sc_agg_w(tab1, ed3)
    pre1 = _tc_pre(x, W1[D:], b1.reshape(1, D))
    h1 = _tc_combine(agg1, w, pre1, W1[:D])

    tab2 = jnp.concatenate([h1[:, :DH], h1[:, DH:]], axis=0)
    (agg2,) = _sc_agg(tab2, ed3)
    pre2 = _tc_pre(h1, W2[D:], b2.reshape(1, D))
    h2 = _tc_combine(agg2, w, pre2, W2[:D])
    return h2
